# baseline (speedup 1.0000x reference)
import functools

import jax
import jax.numpy as jnp
from jax import lax
from jax.experimental import pallas as pl
from jax.experimental.pallas import tpu as pltpu
from jax.experimental.pallas import tpu_sc as plsc

HIDDEN = 128
EMSIZE = 768
CHUNK = 80


def _sc_gather(table, idx, off, ns):
    d = table.shape[1]
    mesh = plsc.VectorSubcoreMesh(core_axis_name="c", subcore_axis_name="s")
    num_cores = mesh.num_cores
    nw = num_cores * mesh.num_subcores
    per_w = ns // nw
    n_ch = per_w // CHUNK

    @functools.partial(
        pl.kernel,
        mesh=mesh,
        out_type=jax.ShapeDtypeStruct((ns, d), jnp.float32),
        scratch_types=[
            pltpu.VMEM((per_w,), jnp.int32),
            pltpu.VMEM((2, CHUNK, d), jnp.float32),
            pltpu.SemaphoreType.DMA,
            pltpu.SemaphoreType.DMA,
        ],
    )
    def gather_kernel(table_hbm, idx_hbm, out_hbm, idx_v, rows_v, sem0, sem1):
        wid = lax.axis_index("s") * num_cores + lax.axis_index("c")
        base = wid * per_w
        pltpu.sync_copy(idx_hbm.at[pl.ds(off + base, per_w)], idx_v)
        sems = (sem0, sem1)

        def fire(c):
            return pltpu.async_copy(
                table_hbm.at[idx_v.at[pl.ds(c * CHUNK, CHUNK)]],
                rows_v.at[c % 2],
                sems[c % 2],
            )

        cp = fire(0)
        for c in range(1, n_ch):
            nxt = fire(c)
            cp.wait()
            pltpu.sync_copy(
                rows_v.at[(c - 1) % 2],
                out_hbm.at[pl.ds(base + (c - 1) * CHUNK, CHUNK)],
            )
            cp = nxt
        cp.wait()
        pltpu.sync_copy(
            rows_v.at[(n_ch - 1) % 2],
            out_hbm.at[pl.ds(base + (n_ch - 1) * CHUNK, CHUNK)],
        )

    return gather_kernel(table, idx)


def _tc_project_slice(acc, x, w, b, row0, n):
    ns = x.shape[0]
    bm = 1024
    assert ns % bm == 0 and row0 % bm == 0
    blk0 = row0 // bm

    def body(*refs):
        x_ref, w_ref, b_ref, o_ref = refs[-4:]
        o_ref[...] = (
            jnp.dot(
                x_ref[...].astype(jnp.bfloat16),
                w_ref[...].astype(jnp.bfloat16),
                preferred_element_type=jnp.float32,
            )
            + b_ref[...]
        )

    in_specs = [
        pl.BlockSpec((bm, HIDDEN), lambda i: (i, 0)),
        pl.BlockSpec((HIDDEN, EMSIZE), lambda i: (0, 0)),
        pl.BlockSpec((1, EMSIZE), lambda i: (0, 0)),
    ]
    args = (x, w, b.reshape(1, EMSIZE))
    aliases = {}
    if acc is not None:
        in_specs = [pl.BlockSpec(memory_space=pl.ANY)] + in_specs
        args = (acc,) + args
        aliases = {0: 0}
    return pl.pallas_call(
        body,
        grid=(ns // bm,),
        in_specs=in_specs,
        out_specs=pl.BlockSpec((bm, EMSIZE), lambda i: (i + blk0, 0)),
        out_shape=jax.ShapeDtypeStruct((n, EMSIZE), jnp.float32),
        input_output_aliases=aliases,
    )(*args)


def kernel(item_embeds, emb_table, W_proj, b_proj):
    batch, hist = item_embeds.shape
    n = batch * hist
    idx = item_embeds.T.reshape(-1)
    n_slices = 1
    ns = n // n_slices
    rows = [_sc_gather(emb_table, idx, s * ns, ns) for s in range(n_slices)]
    out = None
    for s in range(n_slices):
        out = _tc_project_slice(out, rows[s], W_proj, b_proj, s * ns, n)
    return out.reshape(hist, batch, EMSIZE).transpose(1, 0, 2)


# speedup vs baseline: 1.1323x; 1.1323x over previous
import functools

import jax
import jax.numpy as jnp
from jax import lax
from jax.experimental import pallas as pl
from jax.experimental.pallas import tpu as pltpu
from jax.experimental.pallas import tpu_sc as plsc

HIDDEN = 128
EMSIZE = 768
CHUNK = 80


def _sc_gather(table, idx, off, ns):
    d = table.shape[1]
    mesh = plsc.VectorSubcoreMesh(core_axis_name="c", subcore_axis_name="s")
    num_cores = mesh.num_cores
    nw = num_cores * mesh.num_subcores
    per_w = ns // nw
    n_ch = per_w // CHUNK

    @functools.partial(
        pl.kernel,
        mesh=mesh,
        out_type=jax.ShapeDtypeStruct((ns, d), jnp.float32),
        scratch_types=[
            pltpu.VMEM((per_w,), jnp.int32),
            pltpu.VMEM((2, CHUNK, d), jnp.float32),
            pltpu.SemaphoreType.DMA,
            pltpu.SemaphoreType.DMA,
        ],
    )
    def gather_kernel(table_hbm, idx_hbm, out_hbm, idx_v, rows_v, sem0, sem1):
        wid = lax.axis_index("s") * num_cores + lax.axis_index("c")
        base = wid * per_w
        pltpu.sync_copy(idx_hbm.at[pl.ds(off + base, per_w)], idx_v)
        sems = (sem0, sem1)

        def fire(c):
            return pltpu.async_copy(
                table_hbm.at[idx_v.at[pl.ds(c * CHUNK, CHUNK)]],
                rows_v.at[c % 2],
                sems[c % 2],
            )

        cp = fire(0)
        for c in range(1, n_ch):
            nxt = fire(c)
            cp.wait()
            pltpu.sync_copy(
                rows_v.at[(c - 1) % 2],
                out_hbm.at[pl.ds(base + (c - 1) * CHUNK, CHUNK)],
            )
            cp = nxt
        cp.wait()
        pltpu.sync_copy(
            rows_v.at[(n_ch - 1) % 2],
            out_hbm.at[pl.ds(base + (n_ch - 1) * CHUNK, CHUNK)],
        )

    return gather_kernel(table, idx)


def _tc_project_slice(acc, x, w, b, row0, n):
    ns = x.shape[0]
    bm = 2048
    assert ns % bm == 0 and row0 % bm == 0
    blk0 = row0 // bm

    def body(*refs):
        x_ref, w_ref, b_ref, o_ref = refs[-4:]
        o_ref[...] = (
            jnp.dot(x_ref[...], w_ref[...], preferred_element_type=jnp.float32)
            + b_ref[...]
        )

    in_specs = [
        pl.BlockSpec((bm, HIDDEN), lambda i: (i, 0)),
        pl.BlockSpec((HIDDEN, EMSIZE), lambda i: (0, 0)),
        pl.BlockSpec((1, EMSIZE), lambda i: (0, 0)),
    ]
    args = (x, w, b.reshape(1, EMSIZE))
    aliases = {}
    if acc is not None:
        in_specs = [pl.BlockSpec(memory_space=pl.ANY)] + in_specs
        args = (acc,) + args
        aliases = {0: 0}
    return pl.pallas_call(
        body,
        grid=(ns // bm,),
        in_specs=in_specs,
        out_specs=pl.BlockSpec((bm, EMSIZE), lambda i: (i + blk0, 0)),
        out_shape=jax.ShapeDtypeStruct((n, EMSIZE), jnp.float32),
        input_output_aliases=aliases,
    )(*args)


def kernel(item_embeds, emb_table, W_proj, b_proj):
    batch, hist = item_embeds.shape
    n = batch * hist
    idx = item_embeds.T.reshape(-1)
    n_slices = 1
    ns = n // n_slices
    rows = [_sc_gather(emb_table, idx, s * ns, ns) for s in range(n_slices)]
    out = None
    for s in range(n_slices):
        out = _tc_project_slice(out, rows[s], W_proj, b_proj, s * ns, n)
    return out.reshape(hist, batch, EMSIZE).transpose(1, 0, 2)


# speedup vs baseline: 1.1494x; 1.0151x over previous
import functools

import jax
import jax.numpy as jnp
from jax import lax
from jax.experimental import pallas as pl
from jax.experimental.pallas import tpu as pltpu
from jax.experimental.pallas import tpu_sc as plsc

HIDDEN = 128
EMSIZE = 768
CHUNK = 80


def _sc_gather(table, idx, off, ns):
    d = table.shape[1]
    mesh = plsc.VectorSubcoreMesh(core_axis_name="c", subcore_axis_name="s")
    num_cores = mesh.num_cores
    nw = num_cores * mesh.num_subcores
    per_w = ns // nw
    n_ch = per_w // CHUNK

    @functools.partial(
        pl.kernel,
        mesh=mesh,
        out_type=jax.ShapeDtypeStruct((ns, d), jnp.float32),
        scratch_types=[
            pltpu.VMEM((per_w,), jnp.int32),
            pltpu.VMEM((2, CHUNK, d), jnp.float32),
            pltpu.SemaphoreType.DMA,
            pltpu.SemaphoreType.DMA,
        ],
    )
    def gather_kernel(table_hbm, idx_hbm, out_hbm, idx_v, rows_v, sem0, sem1):
        wid = lax.axis_index("s") * num_cores + lax.axis_index("c")
        base = wid * per_w
        pltpu.sync_copy(idx_hbm.at[pl.ds(off + base, per_w)], idx_v)
        sems = (sem0, sem1)

        def fire(c):
            return pltpu.async_copy(
                table_hbm.at[idx_v.at[pl.ds(c * CHUNK, CHUNK)]],
                rows_v.at[c % 2],
                sems[c % 2],
            )

        cp = fire(0)
        for c in range(1, n_ch):
            nxt = fire(c)
            cp.wait()
            pltpu.sync_copy(
                rows_v.at[(c - 1) % 2],
                out_hbm.at[pl.ds(base + (c - 1) * CHUNK, CHUNK)],
            )
            cp = nxt
        cp.wait()
        pltpu.sync_copy(
            rows_v.at[(n_ch - 1) % 2],
            out_hbm.at[pl.ds(base + (n_ch - 1) * CHUNK, CHUNK)],
        )

    return gather_kernel(table, idx)


def _tc_project_slice(acc, x, w, b, row0, n):
    ns = x.shape[0]
    bm = 3200
    assert ns % bm == 0 and row0 % bm == 0
    blk0 = row0 // bm

    def body(*refs):
        x_ref, w_ref, b_ref, o_ref = refs[-4:]
        o_ref[...] = (
            jnp.dot(x_ref[...], w_ref[...], preferred_element_type=jnp.float32)
            + b_ref[...]
        )

    in_specs = [
        pl.BlockSpec((bm, HIDDEN), lambda i: (i, 0)),
        pl.BlockSpec((HIDDEN, EMSIZE), lambda i: (0, 0)),
        pl.BlockSpec((1, EMSIZE), lambda i: (0, 0)),
    ]
    args = (x, w, b.reshape(1, EMSIZE))
    aliases = {}
    if acc is not None:
        in_specs = [pl.BlockSpec(memory_space=pl.ANY)] + in_specs
        args = (acc,) + args
        aliases = {0: 0}
    return pl.pallas_call(
        body,
        grid=(ns // bm,),
        in_specs=in_specs,
        out_specs=pl.BlockSpec((bm, EMSIZE), lambda i: (i + blk0, 0)),
        out_shape=jax.ShapeDtypeStruct((n, EMSIZE), jnp.float32),
        input_output_aliases=aliases,
    )(*args)


def kernel(item_embeds, emb_table, W_proj, b_proj):
    batch, hist = item_embeds.shape
    n = batch * hist
    idx = item_embeds.T.reshape(-1)
    n_slices = 1
    ns = n // n_slices
    rows = [_sc_gather(emb_table, idx, s * ns, ns) for s in range(n_slices)]
    out = None
    for s in range(n_slices):
        out = _tc_project_slice(out, rows[s], W_proj, b_proj, s * ns, n)
    return out.reshape(hist, batch, EMSIZE).transpose(1, 0, 2)


# speedup vs baseline: 1.1551x; 1.0050x over previous
import functools

import jax
import jax.numpy as jnp
from jax import lax
from jax.experimental import pallas as pl
from jax.experimental.pallas import tpu as pltpu
from jax.experimental.pallas import tpu_sc as plsc

HIDDEN = 128
EMSIZE = 768
CHUNK = 80


def _sc_gather(table, idx, off, ns):
    d = table.shape[1]
    mesh = plsc.VectorSubcoreMesh(core_axis_name="c", subcore_axis_name="s")
    num_cores = mesh.num_cores
    nw = num_cores * mesh.num_subcores
    per_w = ns // nw
    n_ch = per_w // CHUNK

    @functools.partial(
        pl.kernel,
        mesh=mesh,
        out_type=jax.ShapeDtypeStruct((ns, d), jnp.float32),
        scratch_types=[
            pltpu.VMEM((per_w,), jnp.int32),
            pltpu.VMEM((2, CHUNK, d), jnp.float32),
            pltpu.SemaphoreType.DMA,
            pltpu.SemaphoreType.DMA,
        ],
    )
    def gather_kernel(table_hbm, idx_hbm, out_hbm, idx_v, rows_v, sem0, sem1):
        wid = lax.axis_index("s") * num_cores + lax.axis_index("c")
        base = wid * per_w
        pltpu.sync_copy(idx_hbm.at[pl.ds(off + base, per_w)], idx_v)
        sems = (sem0, sem1)

        def fire(c):
            return pltpu.async_copy(
                table_hbm.at[idx_v.at[pl.ds(c * CHUNK, CHUNK)]],
                rows_v.at[c % 2],
                sems[c % 2],
            )

        cp = fire(0)
        for c in range(1, n_ch):
            nxt = fire(c)
            cp.wait()
            pltpu.sync_copy(
                rows_v.at[(c - 1) % 2],
                out_hbm.at[pl.ds(base + (c - 1) * CHUNK, CHUNK)],
            )
            cp = nxt
        cp.wait()
        pltpu.sync_copy(
            rows_v.at[(n_ch - 1) % 2],
            out_hbm.at[pl.ds(base + (n_ch - 1) * CHUNK, CHUNK)],
        )

    return gather_kernel(table, idx)


def _tc_project_slice(acc, x, w, b, row0, n):
    ns = x.shape[0]
    bm = 5120
    assert ns % bm == 0 and row0 % bm == 0
    blk0 = row0 // bm

    def body(*refs):
        x_ref, w_ref, b_ref, o_ref = refs[-4:]
        o_ref[...] = (
            jnp.dot(x_ref[...], w_ref[...], preferred_element_type=jnp.float32)
            + b_ref[...]
        )

    in_specs = [
        pl.BlockSpec((bm, HIDDEN), lambda i: (i, 0)),
        pl.BlockSpec((HIDDEN, EMSIZE), lambda i: (0, 0)),
        pl.BlockSpec((1, EMSIZE), lambda i: (0, 0)),
    ]
    args = (x, w, b.reshape(1, EMSIZE))
    aliases = {}
    if acc is not None:
        in_specs = [pl.BlockSpec(memory_space=pl.ANY)] + in_specs
        args = (acc,) + args
        aliases = {0: 0}
    return pl.pallas_call(
        body,
        grid=(ns // bm,),
        in_specs=in_specs,
        out_specs=pl.BlockSpec((bm, EMSIZE), lambda i: (i + blk0, 0)),
        out_shape=jax.ShapeDtypeStruct((n, EMSIZE), jnp.float32),
        input_output_aliases=aliases,
    )(*args)


def kernel(item_embeds, emb_table, W_proj, b_proj):
    batch, hist = item_embeds.shape
    n = batch * hist
    idx = item_embeds.T.reshape(-1)
    n_slices = 1
    ns = n // n_slices
    rows = [_sc_gather(emb_table, idx, s * ns, ns) for s in range(n_slices)]
    out = None
    for s in range(n_slices):
        out = _tc_project_slice(out, rows[s], W_proj, b_proj, s * ns, n)
    return out.reshape(hist, batch, EMSIZE).transpose(1, 0, 2)
